# baseline (speedup 1.0000x reference)
import functools

import jax
import jax.numpy as jnp
from jax.experimental import pallas as pl
from jax.experimental.pallas import tpu as pltpu
from jax.experimental.pallas import tpu_sc as plsc

B = 4
CH = 64
NPTS = 8192
NS = 1024
KNN = 16
SBLK = 128


FPB = 2


def _fps_body(x_ref, xt_ref, idx_ref, qt_ref):
    lane_n = jax.lax.broadcasted_iota(jnp.int32, (1, NPTS), 1)
    pos2d = (jax.lax.broadcasted_iota(jnp.int32, (8, 128), 0) * 128
             + jax.lax.broadcasted_iota(jnp.int32, (8, 128), 1))

    def step(t, carry):
        dist, far, inds = carry
        new_dist, new_far, new_inds = [], [], []
        for b in range(FPB):
            new_inds.append(jnp.where(pos2d == t, far[b], inds[b]))
            row = xt_ref[b, pl.ds(far[b], 1), :]
            qt_ref[b, pl.ds(t, 1), :] = row
            cent = jnp.transpose(row, (1, 0))
            parts = []
            for c0 in range(0, NPTS, 1024):
                diff = x_ref[b, :, c0:c0 + 1024] - cent
                parts.append(jnp.sum(diff * diff, axis=0, keepdims=True))
            d = jnp.concatenate(parts, axis=1)
            db = jnp.minimum(dist[b], d)
            new_dist.append(db)
            m = jnp.max(db)
            new_far.append(jnp.min(jnp.where(db == m, lane_n, NPTS)))
        return tuple(new_dist), tuple(new_far), tuple(new_inds)

    dist0 = tuple(jnp.full((1, NPTS), jnp.inf, dtype=jnp.float32)
                  for _ in range(FPB))
    init = (dist0, tuple(jnp.int32(0) for _ in range(FPB)),
            tuple(jnp.zeros((8, 128), jnp.int32) for _ in range(FPB)))
    _, _, inds = jax.lax.fori_loop(0, NS, step, init)
    for b in range(FPB):
        idx_ref[b] = inds[b]


def _fps_call(x, xt):
    return pl.pallas_call(
        _fps_body,
        grid=(B // FPB,),
        in_specs=[
            pl.BlockSpec((FPB, CH, NPTS), lambda i: (i, 0, 0)),
            pl.BlockSpec((FPB, NPTS, CH), lambda i: (i, 0, 0)),
        ],
        out_specs=[
            pl.BlockSpec((FPB, 8, 128), lambda i: (i, 0, 0)),
            pl.BlockSpec((FPB, NS, CH), lambda i: (i, 0, 0)),
        ],
        out_shape=[
            jax.ShapeDtypeStruct((B, 8, 128), jnp.int32),
            jax.ShapeDtypeStruct((B, NS, CH), jnp.float32),
        ],
        compiler_params=pltpu.CompilerParams(
            dimension_semantics=("parallel",)),
    )(x, xt)


def _knn_body(q_ref, x_ref, kidx_ref):
    xb = x_ref[0]
    qb = q_ref[0]
    p2 = jnp.sum(xb * xb, axis=0, keepdims=True)
    inner = jax.lax.dot_general(
        qb, xb, (((1,), (0,)), ((), ())),
        preferred_element_type=jnp.float32)
    dmat = p2 - 2.0 * inner
    lane_n = jax.lax.broadcasted_iota(jnp.int32, (SBLK, NPTS), 1)
    for j in range(KNN):
        m = jnp.min(dmat, axis=1, keepdims=True)
        am = jnp.min(jnp.where(dmat == m, lane_n, NPTS), axis=1,
                     keepdims=True)
        kidx_ref[0, :, pl.ds(j, 1)] = am
        dmat = jnp.where(lane_n == am, jnp.inf, dmat)


def _knn_call(qt, x):
    return pl.pallas_call(
        _knn_body,
        grid=(B, NS // SBLK),
        in_specs=[
            pl.BlockSpec((1, SBLK, CH), lambda b, s: (b, s, 0)),
            pl.BlockSpec((1, CH, NPTS), lambda b, s: (b, 0, 0)),
        ],
        out_specs=pl.BlockSpec((1, SBLK, KNN), lambda b, s: (b, s, 0)),
        out_shape=jax.ShapeDtypeStruct((B, NS, KNN), jnp.int32),
    )(qt, x)


NROWS = B * NS * KNN
_NW = 32
_RPW = NROWS // _NW
_GCHUNK = 1024


def _sc_gather_call(table, fidx):
    mesh = plsc.VectorSubcoreMesh(core_axis_name="c", subcore_axis_name="s")

    @functools.partial(
        pl.kernel, mesh=mesh,
        out_type=jax.ShapeDtypeStruct((NROWS, CH), jnp.float32),
        scratch_types=[
            pltpu.VMEM((_GCHUNK,), jnp.int32),
            pltpu.VMEM((_GCHUNK, CH), jnp.float32),
            pltpu.SemaphoreType.DMA,
        ],
        compiler_params=pltpu.CompilerParams(use_tc_tiling_on_sc=False),
    )
    def k(table_hbm, idx_hbm, out_hbm, idx_v, rows_v, sem):
        wid = jax.lax.axis_index("s") * 2 + jax.lax.axis_index("c")
        base = wid * _RPW
        for g in range(_RPW // _GCHUNK):
            off = base + g * _GCHUNK
            pltpu.sync_copy(idx_hbm.at[pl.ds(off, _GCHUNK)], idx_v)
            pltpu.async_copy(table_hbm.at[idx_v], rows_v, sem).wait()
            pltpu.sync_copy(rows_v, out_hbm.at[pl.ds(off, _GCHUNK)])

    return k(table, fidx)


def _ec_body(g_ref, qt_ref, w1_ref, b1_ref, w2_ref, b2_ref, out_ref):
    w1 = w1_ref[...]
    w1a = w1[:, :CH]
    w1b = w1[:, CH:]
    wd = w1a - w1b
    qt = qt_ref[0]
    hc = jax.lax.dot_general(
        qt, wd, (((1,), (1,)), ((), ())),
        preferred_element_type=jnp.float32) + b1_ref[...]
    g = g_ref[0, 0]
    hn = jax.lax.dot_general(
        g, w1b, (((1,), (1,)), ((), ())),
        preferred_element_type=jnp.float32)
    h1 = hn + jnp.concatenate([hc] * KNN, axis=0)
    h1 = jnp.where(h1 >= 0, h1, 0.2 * h1)
    h2 = jax.lax.dot_general(
        h1, w2_ref[...], (((1,), (1,)), ((), ())),
        preferred_element_type=jnp.float32) + b2_ref[...]
    h2 = jnp.where(h2 >= 0, h2, 0.2 * h2)
    mx = h2[0:SBLK]
    for j in range(1, KNN):
        mx = jnp.maximum(mx, h2[j * SBLK:(j + 1) * SBLK])
    out_ref[0] = mx


def _ec_call(g4, qt, W1, b1, W2, b2):
    return pl.pallas_call(
        _ec_body,
        grid=(B, NS // SBLK),
        in_specs=[
            pl.BlockSpec((1, 1, KNN * SBLK, CH), lambda b, s: (b, s, 0, 0)),
            pl.BlockSpec((1, SBLK, CH), lambda b, s: (b, s, 0)),
            pl.BlockSpec((CH, 2 * CH), lambda b, s: (0, 0)),
            pl.BlockSpec((1, CH), lambda b, s: (0, 0)),
            pl.BlockSpec((CH, CH), lambda b, s: (0, 0)),
            pl.BlockSpec((1, CH), lambda b, s: (0, 0)),
        ],
        out_specs=pl.BlockSpec((1, SBLK, CH), lambda b, s: (b, s, 0)),
        out_shape=jax.ShapeDtypeStruct((B, NS, CH), jnp.float32),
    )(g4, qt, W1, b1, W2, b2)


def kernel(x, W1, b1, W2, b2):
    xt = jnp.transpose(x, (0, 2, 1))
    idx8, qt = _fps_call(x, xt)
    idx = idx8.reshape(B, NS)
    kidx = _knn_call(qt, x)
    fidx = (kidx.reshape(B, NS // SBLK, SBLK, KNN).transpose(0, 1, 3, 2)
            + (jnp.arange(B, dtype=jnp.int32) * NPTS)[:, None, None, None]
            ).reshape(NROWS)
    g = _sc_gather_call(xt.reshape(B * NPTS, CH), fidx)
    g4 = g.reshape(B, NS // SBLK, KNN * SBLK, CH)
    out_t = _ec_call(g4, qt, W1, b1.reshape(1, CH), W2, b2.reshape(1, CH))
    x_processed = jnp.transpose(out_t, (0, 2, 1))
    return (x_processed, idx)


# speedup vs baseline: 1.1816x; 1.1816x over previous
import functools

import jax
import jax.numpy as jnp
from jax.experimental import pallas as pl
from jax.experimental.pallas import tpu as pltpu
from jax.experimental.pallas import tpu_sc as plsc

B = 4
CH = 64
NPTS = 8192
NS = 1024
KNN = 16
SBLK = 128


def _fps_body(x_ref, xt_ref, idx_ref, qt_ref):
    pos_lin = (jax.lax.broadcasted_iota(jnp.int32, (8, 1024), 0) * 1024
               + jax.lax.broadcasted_iota(jnp.int32, (8, 1024), 1))
    pos2d = (jax.lax.broadcasted_iota(jnp.int32, (8, 128), 0) * 128
             + jax.lax.broadcasted_iota(jnp.int32, (8, 128), 1))

    def step(t, carry):
        dist, far, inds = carry
        new_dist, new_far, new_inds = [], [], []
        for b in range(B):
            new_inds.append(jnp.where(pos2d == t, far[b], inds[b]))
            row = xt_ref[b, pl.ds(far[b], 1), :]
            qt_ref[b, pl.ds(t, 1), :] = row
            cent = jnp.transpose(row, (1, 0))
            parts = []
            for c0 in range(0, NPTS, 1024):
                diff = x_ref[b, :, c0:c0 + 1024] - cent
                parts.append(jnp.sum(diff * diff, axis=0, keepdims=True))
            d = jnp.concatenate(parts, axis=0)
            db = jnp.minimum(dist[b], d)
            new_dist.append(db)
            m = jnp.max(db)
            new_far.append(jnp.min(jnp.where(db == m, pos_lin, NPTS)))
        return tuple(new_dist), tuple(new_far), tuple(new_inds)

    dist0 = tuple(jnp.full((8, NPTS // 8), jnp.inf, dtype=jnp.float32)
                  for _ in range(B))
    init = (dist0, tuple(jnp.int32(0) for _ in range(B)),
            tuple(jnp.zeros((8, 128), jnp.int32) for _ in range(B)))
    _, _, inds = jax.lax.fori_loop(0, NS, step, init)
    for b in range(B):
        idx_ref[b] = inds[b]


def _fps_call(x, xt):
    return pl.pallas_call(
        _fps_body,
        in_specs=[
            pl.BlockSpec((B, CH, NPTS), lambda: (0, 0, 0)),
            pl.BlockSpec((B, NPTS, CH), lambda: (0, 0, 0)),
        ],
        out_specs=[
            pl.BlockSpec((B, 8, 128), lambda: (0, 0, 0)),
            pl.BlockSpec((B, NS, CH), lambda: (0, 0, 0)),
        ],
        out_shape=[
            jax.ShapeDtypeStruct((B, 8, 128), jnp.int32),
            jax.ShapeDtypeStruct((B, NS, CH), jnp.float32),
        ],
    )(x, xt)


def _knn_body(q_ref, x_ref, kidx_ref):
    xb = x_ref[0]
    qb = q_ref[0]
    p2 = jnp.sum(xb * xb, axis=0, keepdims=True)
    inner = jax.lax.dot_general(
        qb, xb, (((1,), (0,)), ((), ())),
        preferred_element_type=jnp.float32)
    dmat = p2 - 2.0 * inner
    lane_n = jax.lax.broadcasted_iota(jnp.int32, (SBLK, NPTS), 1)
    for j in range(KNN):
        m = jnp.min(dmat, axis=1, keepdims=True)
        am = jnp.min(jnp.where(dmat == m, lane_n, NPTS), axis=1,
                     keepdims=True)
        kidx_ref[0, :, pl.ds(j, 1)] = am
        dmat = jnp.where(lane_n == am, jnp.inf, dmat)


def _knn_call(qt, x):
    return pl.pallas_call(
        _knn_body,
        grid=(B, NS // SBLK),
        in_specs=[
            pl.BlockSpec((1, SBLK, CH), lambda b, s: (b, s, 0)),
            pl.BlockSpec((1, CH, NPTS), lambda b, s: (b, 0, 0)),
        ],
        out_specs=pl.BlockSpec((1, SBLK, KNN), lambda b, s: (b, s, 0)),
        out_shape=jax.ShapeDtypeStruct((B, NS, KNN), jnp.int32),
    )(qt, x)


NROWS = B * NS * KNN
_NW = 32
_RPW = NROWS // _NW
_GCHUNK = 1024


def _sc_gather_call(table, fidx):
    mesh = plsc.VectorSubcoreMesh(core_axis_name="c", subcore_axis_name="s")

    @functools.partial(
        pl.kernel, mesh=mesh,
        out_type=jax.ShapeDtypeStruct((NROWS, CH), jnp.float32),
        scratch_types=[
            pltpu.VMEM((_GCHUNK,), jnp.int32),
            pltpu.VMEM((_GCHUNK, CH), jnp.float32),
            pltpu.SemaphoreType.DMA,
        ],
        compiler_params=pltpu.CompilerParams(use_tc_tiling_on_sc=False),
    )
    def k(table_hbm, idx_hbm, out_hbm, idx_v, rows_v, sem):
        wid = jax.lax.axis_index("s") * 2 + jax.lax.axis_index("c")
        base = wid * _RPW
        for g in range(_RPW // _GCHUNK):
            off = base + g * _GCHUNK
            pltpu.sync_copy(idx_hbm.at[pl.ds(off, _GCHUNK)], idx_v)
            pltpu.async_copy(table_hbm.at[idx_v], rows_v, sem).wait()
            pltpu.sync_copy(rows_v, out_hbm.at[pl.ds(off, _GCHUNK)])

    return k(table, fidx)


def _ec_body(g_ref, qt_ref, w1_ref, b1_ref, w2_ref, b2_ref, out_ref):
    w1 = w1_ref[...]
    w1a = w1[:, :CH]
    w1b = w1[:, CH:]
    wd = w1a - w1b
    qt = qt_ref[0]
    hc = jax.lax.dot_general(
        qt, wd, (((1,), (1,)), ((), ())),
        preferred_element_type=jnp.float32) + b1_ref[...]
    g = g_ref[0, 0]
    hn = jax.lax.dot_general(
        g, w1b, (((1,), (1,)), ((), ())),
        preferred_element_type=jnp.float32)
    h1 = hn + jnp.concatenate([hc] * KNN, axis=0)
    h1 = jnp.where(h1 >= 0, h1, 0.2 * h1)
    h2 = jax.lax.dot_general(
        h1, w2_ref[...], (((1,), (1,)), ((), ())),
        preferred_element_type=jnp.float32) + b2_ref[...]
    h2 = jnp.where(h2 >= 0, h2, 0.2 * h2)
    mx = h2[0:SBLK]
    for j in range(1, KNN):
        mx = jnp.maximum(mx, h2[j * SBLK:(j + 1) * SBLK])
    out_ref[0] = mx


def _ec_call(g4, qt, W1, b1, W2, b2):
    return pl.pallas_call(
        _ec_body,
        grid=(B, NS // SBLK),
        in_specs=[
            pl.BlockSpec((1, 1, KNN * SBLK, CH), lambda b, s: (b, s, 0, 0)),
            pl.BlockSpec((1, SBLK, CH), lambda b, s: (b, s, 0)),
            pl.BlockSpec((CH, 2 * CH), lambda b, s: (0, 0)),
            pl.BlockSpec((1, CH), lambda b, s: (0, 0)),
            pl.BlockSpec((CH, CH), lambda b, s: (0, 0)),
            pl.BlockSpec((1, CH), lambda b, s: (0, 0)),
        ],
        out_specs=pl.BlockSpec((1, SBLK, CH), lambda b, s: (b, s, 0)),
        out_shape=jax.ShapeDtypeStruct((B, NS, CH), jnp.float32),
    )(g4, qt, W1, b1, W2, b2)


def kernel(x, W1, b1, W2, b2):
    xt = jnp.transpose(x, (0, 2, 1))
    idx8, qt = _fps_call(x, xt)
    idx = idx8.reshape(B, NS)
    kidx = _knn_call(qt, x)
    fidx = (kidx.reshape(B, NS // SBLK, SBLK, KNN).transpose(0, 1, 3, 2)
            + (jnp.arange(B, dtype=jnp.int32) * NPTS)[:, None, None, None]
            ).reshape(NROWS)
    g = _sc_gather_call(xt.reshape(B * NPTS, CH), fidx)
    g4 = g.reshape(B, NS // SBLK, KNN * SBLK, CH)
    out_t = _ec_call(g4, qt, W1, b1.reshape(1, CH), W2, b2.reshape(1, CH))
    x_processed = jnp.transpose(out_t, (0, 2, 1))
    return (x_processed, idx)
